# baseline (speedup 1.0000x reference)
import functools

import jax
import jax.numpy as jnp
from jax import lax
from jax.experimental import pallas as pl
from jax.experimental.pallas import tpu as pltpu
from jax.experimental.pallas import tpu_sc as plsc

DIM = 1024
NE = 8
INTER = 512
T = 2048
P = 2 * T
BM = 128
NBLK = P // BM + NE
P_PAD = NBLK * BM

NW = 32


def _gate_body(x_ref, gw_ref, sw1_ref, sw3_ref, sw2_ref,
               oh1_ref, oh2_ref, cwm_ref, z_ref):
    x = x_ref[...]
    logits = jnp.dot(x, gw_ref[...], preferred_element_type=jnp.float32)
    scores = jax.nn.softmax(logits, axis=-1)
    lane = lax.broadcasted_iota(jnp.int32, scores.shape, 1)
    m1 = jnp.argmax(scores, axis=-1)
    masked = jnp.where(lane == m1[:, None], -jnp.inf, scores)
    m2 = jnp.argmax(masked, axis=-1)
    oh1 = (lane == m1[:, None]).astype(jnp.float32)
    oh2 = (lane == m2[:, None]).astype(jnp.float32)
    oh1_ref[...] = oh1
    oh2_ref[...] = oh2
    cwm_ref[...] = scores * (oh1 + oh2)

    xb = x.astype(jnp.bfloat16)
    a = jnp.dot(xb, sw1_ref[...], preferred_element_type=jnp.float32)
    b = jnp.dot(xb, sw3_ref[...], preferred_element_type=jnp.float32)
    h = (jax.nn.silu(a) * b).astype(jnp.bfloat16)
    z_ref[...] = jnp.dot(h, sw2_ref[...], preferred_element_type=jnp.float32)


def _gate_shared(xf, gate_w, sw1, sw3, sw2):
    bt = 1024
    return pl.pallas_call(
        _gate_body,
        grid=(T // bt,),
        in_specs=[
            pl.BlockSpec((bt, DIM), lambda t: (t, 0)),
            pl.BlockSpec((DIM, NE), lambda t: (0, 0)),
            pl.BlockSpec((DIM, 2 * INTER), lambda t: (0, 0)),
            pl.BlockSpec((DIM, 2 * INTER), lambda t: (0, 0)),
            pl.BlockSpec((2 * INTER, DIM), lambda t: (0, 0)),
        ],
        out_specs=[
            pl.BlockSpec((bt, NE), lambda t: (t, 0)),
            pl.BlockSpec((bt, NE), lambda t: (t, 0)),
            pl.BlockSpec((bt, NE), lambda t: (t, 0)),
            pl.BlockSpec((bt, DIM), lambda t: (t, 0)),
        ],
        out_shape=[
            jax.ShapeDtypeStruct((T, NE), jnp.float32),
            jax.ShapeDtypeStruct((T, NE), jnp.float32),
            jax.ShapeDtypeStruct((T, NE), jnp.float32),
            jax.ShapeDtypeStruct((T, DIM), jnp.float32),
        ],
    )(xf, gate_w, sw1, sw3, sw2)


_SROWS = T // NW


def _sc_scatter_body(xf_hbm, d0_hbm, d1_hbm, xs_hbm, i0_v, i1_v, buf,
                     ls, s0s, s1s):
    wid = lax.axis_index("s") * 2 + lax.axis_index("c")
    base = wid * _SROWS
    pltpu.sync_copy(d0_hbm.at[pl.ds(base, _SROWS)], i0_v)
    pltpu.sync_copy(d1_hbm.at[pl.ds(base, _SROWS)], i1_v)
    pltpu.async_copy(xf_hbm.at[pl.ds(base, _SROWS)], buf, ls).wait()
    s0 = pltpu.async_copy(buf, xs_hbm.at[i0_v], s0s)
    s1 = pltpu.async_copy(buf, xs_hbm.at[i1_v], s1s)
    s0.wait()
    s1.wait()


def _sc_scatter(xf, dest0, dest1):
    mesh = plsc.VectorSubcoreMesh(core_axis_name="c", subcore_axis_name="s")
    return pl.kernel(
        _sc_scatter_body,
        out_type=jax.ShapeDtypeStruct((P_PAD, DIM), jnp.float32),
        mesh=mesh,
        scratch_types=[
            pltpu.VMEM((_SROWS,), jnp.int32),
            pltpu.VMEM((_SROWS,), jnp.int32),
            pltpu.VMEM((_SROWS, DIM), jnp.float32),
            pltpu.SemaphoreType.DMA,
            pltpu.SemaphoreType.DMA,
            pltpu.SemaphoreType.DMA,
        ],
    )(xf, dest0, dest1)


def _moe_body(be_ref, xs_ref, w1_ref, w3_ref, w2_ref, os_ref):
    del be_ref
    xb = xs_ref[...].astype(jnp.bfloat16)
    a = jnp.dot(xb, w1_ref[0], preferred_element_type=jnp.float32)
    b = jnp.dot(xb, w3_ref[0], preferred_element_type=jnp.float32)
    h = (jax.nn.silu(a) * b).astype(jnp.bfloat16)
    os_ref[...] = jnp.dot(h, w2_ref[0], preferred_element_type=jnp.float32)


def _moe_sparse(block_expert, xs, w1s, w3s, w2s):
    grid_spec = pltpu.PrefetchScalarGridSpec(
        num_scalar_prefetch=1,
        grid=(NBLK,),
        in_specs=[
            pl.BlockSpec((BM, DIM), lambda b, be: (b, 0)),
            pl.BlockSpec((1, DIM, INTER), lambda b, be: (be[b], 0, 0)),
            pl.BlockSpec((1, DIM, INTER), lambda b, be: (be[b], 0, 0)),
            pl.BlockSpec((1, INTER, DIM), lambda b, be: (be[b], 0, 0)),
        ],
        out_specs=pl.BlockSpec((BM, DIM), lambda b, be: (b, 0)),
    )
    return pl.pallas_call(
        _moe_body,
        grid_spec=grid_spec,
        out_shape=jax.ShapeDtypeStruct((P_PAD, DIM), jnp.float32),
    )(block_expert, xs, w1s, w3s, w2s)


_CTOK = T // NW
_CCH = 16
_CN = _CTOK // _CCH


def _sc_combine_body(os_hbm, z_hbm, d0_hbm, d1_hbm, w0_hbm, w1_hbm, y_hbm,
                     i0_v, i1_v, w0_v, w1_v, r0s, r1s, zs, g0s, g1s, zss, sss):
    wid = lax.axis_index("s") * 2 + lax.axis_index("c")
    base = wid * _CTOK
    pltpu.sync_copy(d0_hbm.at[pl.ds(base, _CTOK)], i0_v)
    pltpu.sync_copy(d1_hbm.at[pl.ds(base, _CTOK)], i1_v)
    pltpu.sync_copy(w0_hbm.at[pl.ds(base, _CTOK)], w0_v)
    pltpu.sync_copy(w1_hbm.at[pl.ds(base, _CTOK)], w1_v)

    def issue(c, nb):
        off = base + c * _CCH
        isl = pl.ds(c * _CCH, _CCH)
        return (
            pltpu.async_copy(os_hbm.at[i0_v.at[isl]], r0s[nb], g0s[nb]),
            pltpu.async_copy(os_hbm.at[i1_v.at[isl]], r1s[nb], g1s[nb]),
            pltpu.async_copy(z_hbm.at[pl.ds(off, _CCH)], zs[nb], zss[nb]),
        )

    pend = {0: issue(0, 0)}
    stores = {}
    for c in range(_CN):
        nb = c % 2
        if c + 1 < _CN:
            if c >= 1:
                stores.pop(c - 1).wait()
            pend[c + 1] = issue(c + 1, (c + 1) % 2)
        for cp in pend.pop(c):
            cp.wait()
        wv0 = w0_v[pl.ds(c * _CCH, _CCH)]
        wv1 = w1_v[pl.ds(c * _CCH, _CCH)]
        for t in range(_CCH):
            wa = wv0[t]
            wb = wv1[t]

            def _body(j, _, t=t, nb=nb, wa=wa, wb=wb):
                for u in range(16):
                    s = pl.ds(j * 256 + u * 16, 16)
                    zs[nb][t, s] = (
                        zs[nb][t, s] + wa * r0s[nb][t, s] + wb * r1s[nb][t, s]
                    )
                return 0
            lax.fori_loop(0, DIM // 256, _body, 0)
        off = base + c * _CCH
        stores[c] = pltpu.async_copy(zs[nb], y_hbm.at[pl.ds(off, _CCH)], sss[nb])
    stores.pop(_CN - 2).wait()
    stores.pop(_CN - 1).wait()


def _sc_combine(os, z, dest0, dest1, w0, w1v):
    mesh = plsc.VectorSubcoreMesh(core_axis_name="c", subcore_axis_name="s")
    buf = lambda: [pltpu.VMEM((_CCH, DIM), jnp.float32) for _ in range(2)]
    sem = lambda: [pltpu.SemaphoreType.DMA for _ in range(2)]
    return pl.kernel(
        _sc_combine_body,
        out_type=jax.ShapeDtypeStruct((T, DIM), jnp.float32),
        mesh=mesh,
        scratch_types=[
            pltpu.VMEM((_CTOK,), jnp.int32),
            pltpu.VMEM((_CTOK,), jnp.int32),
            pltpu.VMEM((_CTOK,), jnp.float32),
            pltpu.VMEM((_CTOK,), jnp.float32),
            buf(), buf(), buf(),
            sem(), sem(), sem(), sem(),
        ],
    )(os, z, dest0, dest1, w0, w1v)


def kernel(x, gate_w, w1, w2, w3, sw1, sw2, sw3):
    shape = x.shape
    xf = x.reshape(-1, DIM)

    oh1, oh2, cwm, z = _gate_shared(
        xf,
        gate_w,
        sw1.astype(jnp.bfloat16),
        sw3.astype(jnp.bfloat16),
        sw2.astype(jnp.bfloat16),
    )

    w0 = jnp.sum(cwm * oh1, axis=-1)
    w1v = jnp.sum(cwm * oh2, axis=-1)
    cnt1 = jnp.sum(oh1, axis=0)
    counts = (cnt1 + jnp.sum(oh2, axis=0)).astype(jnp.int32)
    padded = ((counts + BM - 1) // BM) * BM
    ends = jnp.cumsum(padded)
    offsets = (ends - padded).astype(jnp.float32)
    rank1 = jnp.sum((jnp.cumsum(oh1, 0) - oh1) * oh1, axis=-1)
    rank2 = jnp.sum((jnp.cumsum(oh2, 0) - oh2) * oh2, axis=-1)
    off0 = jnp.sum(oh1 * offsets[None, :], axis=-1)
    off1 = jnp.sum(oh2 * (offsets + cnt1)[None, :], axis=-1)
    dest0 = (off0 + rank1).astype(jnp.int32)
    dest1 = (off1 + rank2).astype(jnp.int32)
    b_start = jnp.arange(NBLK, dtype=jnp.int32) * BM
    block_expert = jnp.minimum(
        jnp.sum((b_start[:, None] >= ends[None, :]).astype(jnp.int32), -1), NE - 1
    )

    xs = _sc_scatter(xf, dest0, dest1)
    w1b, w3b, w2b = (w.astype(jnp.bfloat16) for w in (w1, w3, w2))
    os_rows = _moe_sparse(block_expert, xs, w1b, w3b, w2b)

    y = _sc_combine(os_rows, z, dest0, dest1, w0, w1v)
    return y.reshape(shape)


# speedup vs baseline: 1.0411x; 1.0411x over previous
import functools

import jax
import jax.numpy as jnp
from jax import lax
from jax.experimental import pallas as pl
from jax.experimental.pallas import tpu as pltpu
from jax.experimental.pallas import tpu_sc as plsc

DIM = 1024
NE = 8
INTER = 512
T = 2048
P = 2 * T
BM = 128
NBLK = P // BM + NE
P_PAD = NBLK * BM

NW = 32


def _gate_body(x_ref, gw_ref, sw1_ref, sw3_ref, sw2_ref,
               oh1_ref, oh2_ref, cwm_ref, z_ref):
    x = x_ref[...]
    logits = jnp.dot(x, gw_ref[...], preferred_element_type=jnp.float32)
    scores = jax.nn.softmax(logits, axis=-1)
    lane = lax.broadcasted_iota(jnp.int32, scores.shape, 1)
    m1 = jnp.argmax(scores, axis=-1)
    masked = jnp.where(lane == m1[:, None], -jnp.inf, scores)
    m2 = jnp.argmax(masked, axis=-1)
    oh1 = (lane == m1[:, None]).astype(jnp.float32)
    oh2 = (lane == m2[:, None]).astype(jnp.float32)
    oh1_ref[...] = oh1
    oh2_ref[...] = oh2
    cwm_ref[...] = scores * (oh1 + oh2)

    xb = x.astype(jnp.bfloat16)
    a = jnp.dot(xb, sw1_ref[...], preferred_element_type=jnp.float32)
    b = jnp.dot(xb, sw3_ref[...], preferred_element_type=jnp.float32)
    h = (jax.nn.silu(a) * b).astype(jnp.bfloat16)
    z_ref[...] = jnp.dot(h, sw2_ref[...], preferred_element_type=jnp.float32)


def _gate_shared(xf, gate_w, sw1, sw3, sw2):
    bt = 1024
    return pl.pallas_call(
        _gate_body,
        grid=(T // bt,),
        in_specs=[
            pl.BlockSpec((bt, DIM), lambda t: (t, 0)),
            pl.BlockSpec((DIM, NE), lambda t: (0, 0)),
            pl.BlockSpec((DIM, 2 * INTER), lambda t: (0, 0)),
            pl.BlockSpec((DIM, 2 * INTER), lambda t: (0, 0)),
            pl.BlockSpec((2 * INTER, DIM), lambda t: (0, 0)),
        ],
        out_specs=[
            pl.BlockSpec((bt, NE), lambda t: (t, 0)),
            pl.BlockSpec((bt, NE), lambda t: (t, 0)),
            pl.BlockSpec((bt, NE), lambda t: (t, 0)),
            pl.BlockSpec((bt, DIM), lambda t: (t, 0)),
        ],
        out_shape=[
            jax.ShapeDtypeStruct((T, NE), jnp.float32),
            jax.ShapeDtypeStruct((T, NE), jnp.float32),
            jax.ShapeDtypeStruct((T, NE), jnp.float32),
            jax.ShapeDtypeStruct((T, DIM), jnp.float32),
        ],
    )(xf, gate_w, sw1, sw3, sw2)


_SROWS = T // NW


def _sc_scatter_body(xf_hbm, d0_hbm, d1_hbm, xs_hbm, i0_v, i1_v, buf,
                     ls, s0s, s1s):
    wid = lax.axis_index("s") * 2 + lax.axis_index("c")
    base = wid * _SROWS
    pltpu.sync_copy(d0_hbm.at[pl.ds(base, _SROWS)], i0_v)
    pltpu.sync_copy(d1_hbm.at[pl.ds(base, _SROWS)], i1_v)
    pltpu.async_copy(xf_hbm.at[pl.ds(base, _SROWS)], buf, ls).wait()
    s0 = pltpu.async_copy(buf, xs_hbm.at[i0_v], s0s)
    s1 = pltpu.async_copy(buf, xs_hbm.at[i1_v], s1s)
    s0.wait()
    s1.wait()


def _sc_scatter(xf, dest0, dest1):
    mesh = plsc.VectorSubcoreMesh(core_axis_name="c", subcore_axis_name="s")
    return pl.kernel(
        _sc_scatter_body,
        out_type=jax.ShapeDtypeStruct((P_PAD, DIM), jnp.float32),
        mesh=mesh,
        scratch_types=[
            pltpu.VMEM((_SROWS,), jnp.int32),
            pltpu.VMEM((_SROWS,), jnp.int32),
            pltpu.VMEM((_SROWS, DIM), jnp.float32),
            pltpu.SemaphoreType.DMA,
            pltpu.SemaphoreType.DMA,
            pltpu.SemaphoreType.DMA,
        ],
    )(xf, dest0, dest1)


def _moe_body(be_ref, xs_ref, w1_ref, w3_ref, w2_ref, os_ref):
    del be_ref
    xb = xs_ref[...].astype(jnp.bfloat16)
    a = jnp.dot(xb, w1_ref[0], preferred_element_type=jnp.float32)
    b = jnp.dot(xb, w3_ref[0], preferred_element_type=jnp.float32)
    h = (jax.nn.silu(a) * b).astype(jnp.bfloat16)
    os_ref[...] = jnp.dot(h, w2_ref[0], preferred_element_type=jnp.float32)


def _moe_sparse(block_expert, xs, w1s, w3s, w2s):
    grid_spec = pltpu.PrefetchScalarGridSpec(
        num_scalar_prefetch=1,
        grid=(NBLK,),
        in_specs=[
            pl.BlockSpec((BM, DIM), lambda b, be: (b, 0)),
            pl.BlockSpec((1, DIM, INTER), lambda b, be: (be[b], 0, 0)),
            pl.BlockSpec((1, DIM, INTER), lambda b, be: (be[b], 0, 0)),
            pl.BlockSpec((1, INTER, DIM), lambda b, be: (be[b], 0, 0)),
        ],
        out_specs=pl.BlockSpec((BM, DIM), lambda b, be: (b, 0)),
    )
    return pl.pallas_call(
        _moe_body,
        grid_spec=grid_spec,
        out_shape=jax.ShapeDtypeStruct((P_PAD, DIM), jnp.float32),
    )(block_expert, xs, w1s, w3s, w2s)


_CTOK = T // NW
_CCH = 16
_CN = _CTOK // _CCH


def _sc_combine_body(os_hbm, z_hbm, d0_hbm, d1_hbm, w0_hbm, w1_hbm, y_hbm,
                     i0_v, i1_v, w0_v, w1_v, r0s, r1s, zs, g0s, g1s, zss, sss):
    wid = lax.axis_index("s") * 2 + lax.axis_index("c")
    base = wid * _CTOK
    pltpu.sync_copy(d0_hbm.at[pl.ds(base, _CTOK)], i0_v)
    pltpu.sync_copy(d1_hbm.at[pl.ds(base, _CTOK)], i1_v)
    pltpu.sync_copy(w0_hbm.at[pl.ds(base, _CTOK)], w0_v)
    pltpu.sync_copy(w1_hbm.at[pl.ds(base, _CTOK)], w1_v)

    def issue(c, nb):
        off = base + c * _CCH
        isl = pl.ds(c * _CCH, _CCH)
        return (
            pltpu.async_copy(os_hbm.at[i0_v.at[isl]], r0s[nb], g0s[nb]),
            pltpu.async_copy(os_hbm.at[i1_v.at[isl]], r1s[nb], g1s[nb]),
            pltpu.async_copy(z_hbm.at[pl.ds(off, _CCH)], zs[nb], zss[nb]),
        )

    pend = {0: issue(0, 0)}
    stores = {}
    for c in range(_CN):
        nb = c % 2
        if c + 1 < _CN:
            if c >= 1:
                stores.pop(c - 1).wait()
            pend[c + 1] = issue(c + 1, (c + 1) % 2)
        for cp in pend.pop(c):
            cp.wait()
        wv0 = w0_v[pl.ds(c * _CCH, _CCH)]
        wv1 = w1_v[pl.ds(c * _CCH, _CCH)]
        for t in range(_CCH):
            wa = wv0[t]
            wb = wv1[t]

            def _body(j, _, t=t, nb=nb, wa=wa, wb=wb):
                for u in range(8):
                    s = pl.ds(j * 128 + u * 16, 16)
                    zs[nb][t, s] = (
                        zs[nb][t, s] + wa * r0s[nb][t, s] + wb * r1s[nb][t, s]
                    )
                return 0
            lax.fori_loop(0, DIM // 128, _body, 0)
        off = base + c * _CCH
        stores[c] = pltpu.async_copy(zs[nb], y_hbm.at[pl.ds(off, _CCH)], sss[nb])
    stores.pop(_CN - 2).wait()
    stores.pop(_CN - 1).wait()


def _sc_combine(os, z, dest0, dest1, w0, w1v):
    mesh = plsc.VectorSubcoreMesh(core_axis_name="c", subcore_axis_name="s")
    buf = lambda: [pltpu.VMEM((_CCH, DIM), jnp.float32) for _ in range(2)]
    sem = lambda: [pltpu.SemaphoreType.DMA for _ in range(2)]
    return pl.kernel(
        _sc_combine_body,
        out_type=jax.ShapeDtypeStruct((T, DIM), jnp.float32),
        mesh=mesh,
        scratch_types=[
            pltpu.VMEM((_CTOK,), jnp.int32),
            pltpu.VMEM((_CTOK,), jnp.int32),
            pltpu.VMEM((_CTOK,), jnp.float32),
            pltpu.VMEM((_CTOK,), jnp.float32),
            buf(), buf(), buf(),
            sem(), sem(), sem(), sem(),
        ],
    )(os, z, dest0, dest1, w0, w1v)


def kernel(x, gate_w, w1, w2, w3, sw1, sw2, sw3):
    shape = x.shape
    xf = x.reshape(-1, DIM)

    oh1, oh2, cwm, z = _gate_shared(
        xf,
        gate_w,
        sw1.astype(jnp.bfloat16),
        sw3.astype(jnp.bfloat16),
        sw2.astype(jnp.bfloat16),
    )

    w0 = jnp.sum(cwm * oh1, axis=-1)
    w1v = jnp.sum(cwm * oh2, axis=-1)
    cnt1 = jnp.sum(oh1, axis=0)
    counts = (cnt1 + jnp.sum(oh2, axis=0)).astype(jnp.int32)
    padded = ((counts + BM - 1) // BM) * BM
    ends = jnp.cumsum(padded)
    offsets = (ends - padded).astype(jnp.float32)
    rank1 = jnp.sum((jnp.cumsum(oh1, 0) - oh1) * oh1, axis=-1)
    rank2 = jnp.sum((jnp.cumsum(oh2, 0) - oh2) * oh2, axis=-1)
    off0 = jnp.sum(oh1 * offsets[None, :], axis=-1)
    off1 = jnp.sum(oh2 * (offsets + cnt1)[None, :], axis=-1)
    dest0 = (off0 + rank1).astype(jnp.int32)
    dest1 = (off1 + rank2).astype(jnp.int32)
    b_start = jnp.arange(NBLK, dtype=jnp.int32) * BM
    block_expert = jnp.minimum(
        jnp.sum((b_start[:, None] >= ends[None, :]).astype(jnp.int32), -1), NE - 1
    )

    xs = _sc_scatter(xf, dest0, dest1)
    w1b, w3b, w2b = (w.astype(jnp.bfloat16) for w in (w1, w3, w2))
    os_rows = _moe_sparse(block_expert, xs, w1b, w3b, w2b)

    y = _sc_combine(os_rows, z, dest0, dest1, w0, w1v)
    return y.reshape(shape)
